# baseline (speedup 1.0000x reference)
import functools

import jax
import jax.numpy as jnp
from jax import lax
from jax.experimental import pallas as pl
from jax.experimental.pallas import tpu as pltpu
from jax.experimental.pallas import tpu_sc as plsc

DIM = 32
L = 16

_info = plsc.get_sparse_core_info()
NC, NS = _info.num_cores, _info.num_subcores
NW = NC * NS

CH = 128
DEPTH = 4
LEAD = 2


def _gather_kernel(B, H, S, n_streams):
    mesh = plsc.VectorSubcoreMesh(core_axis_name="c", subcore_axis_name="s")

    @functools.partial(
        pl.kernel,
        mesh=mesh,
        out_type=jax.ShapeDtypeStruct((B, H, DIM, S), jnp.float32),
        scratch_types=[
            pltpu.VMEM((n_streams, CH), jnp.int32),
            pltpu.VMEM((n_streams, CH), jnp.int32),
            pltpu.VMEM((DEPTH, CH, 128), jnp.float32),
            pltpu.VMEM((2, DIM, CH), jnp.float32),
            pltpu.VMEM((L,), jnp.int32),
            pltpu.SemaphoreType.DMA,
            pltpu.SemaphoreType.DMA,
        ],
        compiler_params=pltpu.CompilerParams(
            use_tc_tiling_on_sc=True, needs_layout_passes=False),
    )
    def body(ids_hbm, off_hbm, table_hbm, out_hbm, idx_v, row_v, rows_v,
             stage_v, off_v, g_sem, w_sem):
        wid = lax.axis_index("s") * NC + lax.axis_index("c")
        b = wid // H
        h = wid % H
        pltpu.sync_copy(off_hbm.at[h], off_v)
        pltpu.sync_copy(ids_hbm.at[b, h], idx_v)
        off = off_v[...]

        def shift(j, carry):
            for i in range(CH // L):
                s = pl.ds(i * L, L)
                r = idx_v[j, s] + off
                idx_v[j, s] = r
                row_v[j, s] = r >> 2
            return carry

        lax.fori_loop(0, n_streams, shift, 0)

        lanes = lax.iota(jnp.int32, L)

        def fire(j):
            pltpu.async_copy(
                table_hbm.at[row_v.at[j]], rows_v.at[j & (DEPTH - 1)], g_sem)

        for j in range(LEAD):
            fire(j)

        def step(j, carry):
            @pl.when(j + LEAD < n_streams)
            def _():
                fire(j + LEAD)

            rows = rows_v.at[j & (DEPTH - 1)]
            pltpu.make_async_copy(table_hbm.at[row_v.at[0]], rows, g_sem).wait()
            stage = stage_v.at[j & 1]

            @pl.when(j >= 2)
            def _():
                pltpu.make_async_copy(
                    stage, out_hbm.at[b, h, :, pl.ds(0, CH)], w_sem).wait()

            for d in range(DIM):
                def col(i, c):
                    s = pl.ds(i * L, L)
                    win = (idx_v[j, s] & 3) * DIM
                    stage[d, s] = plsc.load_gather(rows, [lanes + i * L, win + d])
                    return c
                lax.fori_loop(0, CH // L, col, 0)

            pltpu.async_copy(
                stage, out_hbm.at[b, h, :, pl.ds(j * CH, CH)], w_sem)
            return carry

        lax.fori_loop(0, n_streams, step, 0)
        for _ in range(2):
            pltpu.make_async_copy(
                stage_v.at[0], out_hbm.at[b, h, :, pl.ds(0, CH)], w_sem).wait()

    return body


def kernel(input_ids, offsets, W):
    B, S, H = input_ids.shape
    R, _ = W.shape
    n_streams = S // CH
    ids_t = jnp.transpose(input_ids, (0, 2, 1)).reshape(B, H, S // CH, CH)
    table = W.reshape(R * DIM // 128, 128)
    offs_b = jnp.broadcast_to(offsets[:, None], (H, L))
    out_phys = _gather_kernel(B, H, S, n_streams)(ids_t, offs_b, table)
    return jnp.transpose(out_phys, (0, 3, 1, 2))


# speedup vs baseline: 1.1351x; 1.1351x over previous
import functools

import jax
import jax.numpy as jnp
from jax import lax
from jax.experimental import pallas as pl
from jax.experimental.pallas import tpu as pltpu
from jax.experimental.pallas import tpu_sc as plsc

DIM = 32
L = 16

_info = plsc.get_sparse_core_info()
NC, NS = _info.num_cores, _info.num_subcores
NW = NC * NS

CH = 128
DEPTH = 4
LEAD = 2


def _gather_kernel(B, H, S, n_streams):
    mesh = plsc.VectorSubcoreMesh(core_axis_name="c", subcore_axis_name="s")

    @functools.partial(
        pl.kernel,
        mesh=mesh,
        out_type=jax.ShapeDtypeStruct((B, H, DIM, S), jnp.float32),
        scratch_types=[
            pltpu.VMEM((n_streams, CH), jnp.int32),
            pltpu.VMEM((DEPTH, CH, 128), jnp.float32),
            pltpu.VMEM((2, DIM, CH), jnp.float32),
            pltpu.VMEM((L,), jnp.int32),
            pltpu.SemaphoreType.DMA,
            pltpu.SemaphoreType.DMA,
        ],
        compiler_params=pltpu.CompilerParams(
            use_tc_tiling_on_sc=True, needs_layout_passes=False),
    )
    def body(ids_hbm, off_hbm, table_hbm, out_hbm, idx_v, rows_v,
             stage_v, off_v, g_sem, w_sem):
        wid = lax.axis_index("s") * NC + lax.axis_index("c")
        b = wid // H
        h = wid % H
        pltpu.sync_copy(off_hbm.at[h], off_v)
        pltpu.sync_copy(ids_hbm.at[b, h], idx_v)
        off = off_v[...]

        def shift(j, carry):
            for i in range(CH // L):
                s = pl.ds(i * L, L)
                idx_v[j, s] = idx_v[j, s] + off
            return carry

        lax.fori_loop(0, n_streams, shift, 0)

        lanes = lax.iota(jnp.int32, L)
        dcols = [jnp.full((L,), d, jnp.int32) for d in range(DIM)]

        def fire(j):
            pltpu.async_copy(
                table_hbm.at[idx_v.at[j]], rows_v.at[j & (DEPTH - 1)], g_sem)

        for j in range(LEAD):
            fire(j)

        def step(j, carry):
            @pl.when(j + LEAD < n_streams)
            def _():
                fire(j + LEAD)

            rows = rows_v.at[j & (DEPTH - 1)]
            pltpu.make_async_copy(table_hbm.at[idx_v.at[0]], rows, g_sem).wait()
            stage = stage_v.at[j & 1]

            @pl.when(j >= 2)
            def _():
                pltpu.make_async_copy(
                    stage, out_hbm.at[b, h, :, pl.ds(0, CH)], w_sem).wait()

            def col(i, c):
                pos = lanes + i * L
                s = pl.ds(i * L, L)
                for d in range(DIM):
                    stage[d, s] = plsc.load_gather(rows, [pos, dcols[d]])
                return c

            lax.fori_loop(0, CH // L, col, 0)
            pltpu.async_copy(
                stage, out_hbm.at[b, h, :, pl.ds(j * CH, CH)], w_sem)
            return carry

        lax.fori_loop(0, n_streams, step, 0)
        for _ in range(2):
            pltpu.make_async_copy(
                stage_v.at[0], out_hbm.at[b, h, :, pl.ds(0, CH)], w_sem).wait()

    return body


def kernel(input_ids, offsets, W):
    B, S, H = input_ids.shape
    n_streams = S // CH
    ids_t = jnp.transpose(input_ids, (0, 2, 1)).reshape(B, H, S // CH, CH)
    table = jnp.pad(W, ((0, 0), (0, 128 - DIM)))
    offs_b = jnp.broadcast_to(offsets[:, None], (H, L))
    out_phys = _gather_kernel(B, H, S, n_streams)(ids_t, offs_b, table)
    return jnp.transpose(out_phys, (0, 3, 1, 2))
